# baseline (speedup 1.0000x reference)
import functools

import jax
import jax.numpy as jnp
from jax.experimental import pallas as pl
from jax.experimental.pallas import tpu as pltpu
from jax.experimental.pallas import tpu_sc as plsc

KNN = 20
EPSV = 1e-5
_HI = jax.lax.Precision.HIGHEST
F32 = jnp.float32


def _lrelu(v):
    return jnp.where(v >= 0, v, 0.2 * v)


def _mm(a, b):
    return jax.lax.dot_general(a, b, (((1,), (0,)), ((), ())),
                               precision=_HI, preferred_element_type=F32)


def _mmt(a, b):
    return jax.lax.dot_general(a, b, (((1,), (1,)), ((), ())),
                               precision=_HI, preferred_element_type=F32)


BF16 = jnp.bfloat16


def _mmb(a, b):
    return jax.lax.dot_general(a.astype(BF16), b.astype(BF16),
                               (((1,), (0,)), ((), ())),
                               preferred_element_type=F32)


def _mmtb(a, b):
    return jax.lax.dot_general(a.astype(BF16), b.astype(BF16),
                               (((1,), (1,)), ((), ())),
                               preferred_element_type=F32)


def _dist(xc, scol, srow, m, n):
    g = _mmtb(xc[:m], xc[:n])
    return (scol[:m] - 2.0 * g) + srow[:, :n]


def _mmbb(a, b):
    return jax.lax.dot_general(a, b, (((1,), (0,)), ((), ())),
                               preferred_element_type=F32)


def _knn_maxpool(d, feats):
    M, N = d.shape
    C = feats.shape[1]
    h1 = feats.astype(BF16)
    r1 = feats - h1.astype(F32)
    h2 = r1.astype(BF16)
    h3 = (r1 - h2.astype(F32)).astype(BF16)
    iota = jax.lax.broadcasted_iota(jnp.int32, (M, N), 1)
    pooled = jnp.full((M, C), -jnp.inf, F32)
    for _ in range(KNN):
        rowmin = jnp.min(d, axis=1, keepdims=True)
        am = jnp.min(jnp.where(d == rowmin, iota, N), axis=1, keepdims=True)
        onehot = iota == am
        d = jnp.where(onehot, F32(1e30), d)
        ohb = onehot.astype(BF16)
        sel = (_mmbb(ohb, h1) + _mmbb(ohb, h2)) + _mmbb(ohb, h3)
        pooled = jnp.maximum(pooled, sel)
    return pooled


def _norm(y, st, gb, cnt):
    m = st[0:1, :] / cnt
    v = st[1:2, :] / cnt - m * m
    sc = gb[0:1, :] * jax.lax.rsqrt(v + EPSV)
    return _lrelu((y - m) * sc + gb[1:2, :])


def _stat_rows(y):
    s0 = jnp.sum(y, axis=0, keepdims=True)
    s1 = jnp.sum(y * y, axis=0, keepdims=True)
    rows = jax.lax.broadcasted_iota(jnp.int32, (8, y.shape[1]), 0)
    return jnp.where(rows == 0, jnp.broadcast_to(s0, (8, y.shape[1])),
                     jnp.where(rows == 1, jnp.broadcast_to(s1, (8, y.shape[1])),
                               F32(0.0)))


def _accum_stats(b, nb, upd, st_ref, acc_ref):
    @pl.when(b == 0)
    def _():
        acc_ref[...] = jnp.zeros_like(acc_ref)
    acc_ref[...] += upd

    @pl.when(b == nb - 1)
    def _():
        st_ref[...] = acc_ref[...]


def _c1_body(nb, xd_ref, w_ref, y_ref, st_ref, acc_ref):
    b = pl.program_id(0)
    y = _mmb(xd_ref[0], w_ref[...])
    y_ref[0] = y
    _accum_stats(b, nb, _stat_rows(y), st_ref, acc_ref)


def _mid_body(nb, npts_in, npool, nsub, y_ref, st_ref, xc_ref, scol_ref,
              srow_ref, w_ref, gb_ref, yo_ref, xs_ref, st2_ref, acc_ref):
    b = pl.program_id(0)
    cnt = F32(nb * npts_in)
    f = _norm(y_ref[0], st_ref[...], gb_ref[...], cnt)
    xc = xc_ref[0]
    scol = scol_ref[0]
    srow = srow_ref[0]
    d = _dist(xc, scol, srow, npool, npts_in)
    fp = _knn_maxpool(d, f)
    ds = _dist(xc, scol, srow, nsub, npool)
    xs_ref[0] = _knn_maxpool(ds, fp)
    yo = _mmb(fp, w_ref[...])
    yo_ref[0] = yo
    _accum_stats(b, nb, _stat_rows(yo), st2_ref, acc_ref)


def _c2a_body(nb, npts, npool, y_ref, st_ref, xc_ref, scol_ref, srow_ref,
              gb_ref, f_ref, idx_ref):
    b = pl.program_id(0)
    cnt = F32(nb * npts)
    f = _norm(y_ref[0], st_ref[...], gb_ref[...], cnt)
    f_ref[0] = jnp.concatenate(
        [f, jnp.zeros((f.shape[0], 128 - f.shape[1]), F32)], axis=1)
    d = _dist(xc_ref[0], scol_ref[0], srow_ref[0], npool, npts)
    base = b * npts
    iota = jax.lax.broadcasted_iota(jnp.int32, (npool, npts), 1)
    for t in range(KNN):
        rowmin = jnp.min(d, axis=1, keepdims=True)
        am = jnp.min(jnp.where(d == rowmin, iota, npts), axis=1, keepdims=True)
        idx_ref[0, :, t:t + 1] = am + base
        d = jnp.where(iota == am, F32(1e30), d)


def _sc_gather_body(per_tile, chunk, table_ref, idx_ref, out_ref,
                    idx_v, rows_v, sem):
    c = jax.lax.axis_index("c")
    s = jax.lax.axis_index("s")
    base = (s * 2 + c) * per_tile

    @pl.loop(0, per_tile, step=chunk)
    def _(off):
        pltpu.sync_copy(idx_ref.at[pl.ds(base + off, chunk)], idx_v)
        pltpu.async_copy(table_ref.at[idx_v], rows_v, sem).wait()
        pltpu.sync_copy(rows_v, out_ref.at[pl.ds(base + off, chunk)])


def _c2c_body(nb, npool, nsub, cin, cout, g_ref, xc_ref, scol_ref, srow_ref,
              w_ref, yo_ref, xs_ref, st2_ref, acc_ref):
    b = pl.program_id(0)
    g = g_ref[0]
    fp = jnp.max(g.reshape(npool, KNN, 128), axis=1)[:, :cin]
    ds = _dist(xc_ref[0], scol_ref[0], srow_ref[0], nsub, npool)
    xs_ref[0] = _knn_maxpool(ds, fp)
    yo = _mmb(fp, w_ref[...])
    yo_ref[0] = yo
    _accum_stats(b, nb, _stat_rows(yo), st2_ref, acc_ref)


def _c5_body(nb, npts_in, nsub, y_ref, st_ref, xc_ref, scol_ref, srow_ref,
             x1_ref, x2_ref, x3_ref, w_ref, gb_ref, yo_ref, st2_ref, acc_ref):
    b = pl.program_id(0)
    cnt = F32(nb * npts_in)
    f4 = _norm(y_ref[0], st_ref[...], gb_ref[...], cnt)
    d4 = _dist(xc_ref[0], scol_ref[0], srow_ref[0], nsub, npts_in)
    fp4 = _knn_maxpool(d4, f4)
    h = jnp.concatenate([x1_ref[0], x2_ref[0], x3_ref[0], fp4], axis=1)
    yo = _mmb(h, w_ref[...])
    yo_ref[0] = yo
    _accum_stats(b, nb, _stat_rows(yo), st2_ref, acc_ref)


def _head_body(nb, nsub, y5_ref, st_ref, gb5_ref, l1_ref, gb6_ref, l2_ref,
               b2_ref, gb7_ref, l3_ref, b3_ref, o_ref):
    cnt = F32(nb * nsub)
    y5 = y5_ref[...]
    st = st_ref[...]
    m = (st[0:1, :] / cnt)[None]
    v = (st[1:2, :] / cnt)[None] - m * m
    sc = gb5_ref[0:1, :][None] * jax.lax.rsqrt(v + EPSV)
    f5 = _lrelu((y5 - m) * sc + gb5_ref[1:2, :][None])
    h1 = jnp.max(f5, axis=1)
    h2 = jnp.mean(f5, axis=1)
    h = jnp.concatenate([h1, h2], axis=1)

    t = _mmb(h, l1_ref[...])
    mb = jnp.mean(t, axis=0, keepdims=True)
    vb = jnp.mean(t * t, axis=0, keepdims=True) - mb * mb
    t = _lrelu(gb6_ref[0:1, :] * (t - mb) * jax.lax.rsqrt(vb + EPSV)
               + gb6_ref[1:2, :])

    t = _mmb(t, l2_ref[...]) + b2_ref[...]
    mb = jnp.mean(t, axis=0, keepdims=True)
    vb = jnp.mean(t * t, axis=0, keepdims=True) - mb * mb
    t = _lrelu(gb7_ref[0:1, :] * (t - mb) * jax.lax.rsqrt(vb + EPSV)
               + gb7_ref[1:2, :])

    o_ref[...] = _mmb(t, l3_ref[...]) + b3_ref[...]


def _pack_gb(g, be):
    z = jnp.zeros((8, g.shape[0]), F32)
    return z.at[0].set(g).at[1].set(be)


def kernel(x, W1, W2, W3, W4, W5, L1, L2, b2, L3, b3,
           g1, be1, g2, be2, g3, be3, g4, be4, g5, be5, g6, be6, g7, be7):
    B, _, N = x.shape
    xt = jnp.transpose(x, (0, 2, 1))
    scol = jnp.sum(xt * xt, axis=2, keepdims=True)
    srow = jnp.transpose(scol, (0, 2, 1))
    xc = jnp.concatenate([xt, jnp.zeros((B, N, 5), F32)], axis=2)

    w1p = jnp.zeros((8, 64), F32).at[:3].set(W1.T)
    gbs = [_pack_gb(g1, be1), _pack_gb(g2, be2), _pack_gb(g3, be3),
           _pack_gb(g4, be4), _pack_gb(g5, be5), _pack_gb(g6, be6),
           _pack_gb(g7, be7)]

    cparams = pltpu.CompilerParams(dimension_semantics=("arbitrary",))

    per_b = lambda *dims: pl.BlockSpec((1,) + dims, lambda b: (b, 0, 0))
    const2 = lambda r, c: pl.BlockSpec((r, c), lambda b: (0, 0))

    y1, st1 = pl.pallas_call(
        functools.partial(_c1_body, B),
        grid=(B,),
        in_specs=[per_b(N, 8), const2(8, 64)],
        out_specs=[per_b(N, 64), const2(8, 64)],
        out_shape=[jax.ShapeDtypeStruct((B, N, 64), F32),
                   jax.ShapeDtypeStruct((8, 64), F32)],
        scratch_shapes=[pltpu.VMEM((8, 64), F32)],
        compiler_params=cparams,
    )(xc, w1p)

    def mid_call(yin, stin, cin, cout, npts_in, npool, nsub, wt, gb):
        return pl.pallas_call(
            functools.partial(_mid_body, B, npts_in, npool, nsub),
            grid=(B,),
            in_specs=[per_b(npts_in, cin), const2(8, cin),
                      per_b(npts_in, 8), per_b(npts_in, 1),
                      pl.BlockSpec((1, 1, npts_in), lambda b: (b, 0, 0)),
                      const2(cin, cout), const2(8, cin)],

            out_specs=[per_b(npool, cout), per_b(nsub, cin), const2(8, cout)],
            out_shape=[jax.ShapeDtypeStruct((B, npool, cout), F32),
                       jax.ShapeDtypeStruct((B, nsub, cin), F32),
                       jax.ShapeDtypeStruct((8, cout), F32)],
            scratch_shapes=[pltpu.VMEM((8, cout), F32)],
            compiler_params=cparams,
        )(yin, stin, xc[:, :npts_in], scol[:, :npts_in],
          srow[:, :, :npts_in], wt, gb)

    npool1 = N // 4
    hb = B // 2

    def stage1_half(y1h, xch, scolh, srowh):
        f1h, idxh = pl.pallas_call(
            functools.partial(_c2a_body, B, N, npool1),
            grid=(hb,),
            in_specs=[per_b(N, 64), const2(8, 64),
                      per_b(N, 8), per_b(N, 1),
                      pl.BlockSpec((1, 1, N), lambda b: (b, 0, 0)),
                      const2(8, 64)],
            out_specs=[per_b(N, 128), per_b(npool1, KNN)],
            out_shape=[jax.ShapeDtypeStruct((hb, N, 128), F32),
                       jax.ShapeDtypeStruct((hb, npool1, KNN), jnp.int32)],
            compiler_params=cparams,
        )(y1h, st1, xch, scolh, srowh, gbs[0])

        tot = hb * npool1 * KNN
        per_tile = tot // 32
        chunk = 256
        gath = pl.kernel(
            functools.partial(_sc_gather_body, per_tile, chunk),
            out_type=jax.ShapeDtypeStruct((tot, 128), F32),
            mesh=plsc.VectorSubcoreMesh(core_axis_name="c",
                                        subcore_axis_name="s"),
            scratch_types=[pltpu.VMEM((chunk,), jnp.int32),
                           pltpu.VMEM((chunk, 128), F32),
                           pltpu.SemaphoreType.DMA],
        )(f1h.reshape(hb * N, 128), idxh.reshape(tot))
        return gath

    def stage1_back(gath, xch, scolh, srowh):
        return pl.pallas_call(
            functools.partial(_c2c_body, hb, npool1, N // 64, 64, 64),
            grid=(hb,),
            in_specs=[per_b(npool1 * KNN, 128),
                      per_b(npool1, 8), per_b(npool1, 1),
                      pl.BlockSpec((1, 1, npool1), lambda b: (b, 0, 0)),
                      const2(64, 64)],
            out_specs=[per_b(npool1, 64), per_b(N // 64, 64), const2(8, 64)],
            out_shape=[jax.ShapeDtypeStruct((hb, npool1, 64), F32),
                       jax.ShapeDtypeStruct((hb, N // 64, 64), F32),
                       jax.ShapeDtypeStruct((8, 64), F32)],
            scratch_shapes=[pltpu.VMEM((8, 64), F32)],
            compiler_params=cparams,
        )(gath.reshape(hb, npool1 * KNN, 128), xch,
          scolh, srowh, W2.T)

    g_h1 = stage1_half(y1[:hb], xc[:hb], scol[:hb], srow[:hb])
    g_h2 = stage1_half(y1[hb:], xc[hb:], scol[hb:], srow[hb:])
    y2a, x1a, st2a = stage1_back(g_h1, xc[:hb, :npool1],
                                 scol[:hb, :npool1], srow[:hb, :, :npool1])
    y2b, x1b, st2b = stage1_back(g_h2, xc[hb:, :npool1],
                                 scol[hb:, :npool1], srow[hb:, :, :npool1])
    y2 = jnp.concatenate([y2a, y2b], axis=0)
    x1 = jnp.concatenate([x1a, x1b], axis=0)
    st2 = st2a + st2b
    y3, x2, st3 = mid_call(y2, st2, 64, 128, N // 4, N // 16, N // 64,
                           W3.T, gbs[1])
    y4, x3, st4 = mid_call(y3, st3, 128, 256, N // 16, N // 32, N // 64,
                           W4.T, gbs[2])

    p5 = N // 32
    ns = N // 64
    y5, st5 = pl.pallas_call(
        functools.partial(_c5_body, B, p5, ns),
        grid=(B,),
        in_specs=[per_b(p5, 256), const2(8, 256),
                  per_b(p5, 8), per_b(p5, 1),
                  pl.BlockSpec((1, 1, p5), lambda b: (b, 0, 0)),
                  per_b(ns, 64), per_b(ns, 64), per_b(ns, 128),
                  const2(512, 1024), const2(8, 256)],
        out_specs=[per_b(ns, 1024), const2(8, 1024)],
        out_shape=[jax.ShapeDtypeStruct((B, ns, 1024), F32),
                   jax.ShapeDtypeStruct((8, 1024), F32)],
        scratch_shapes=[pltpu.VMEM((8, 1024), F32)],
        compiler_params=cparams,
    )(y4, st4, xc[:, :p5], scol[:, :p5], srow[:, :, :p5],
      x1, x2, x3, W5.T, gbs[3])

    out = pl.pallas_call(
        functools.partial(_head_body, B, ns),
        in_specs=[pl.BlockSpec((B, ns, 1024), lambda: (0, 0, 0)),
                  pl.BlockSpec((8, 1024), lambda: (0, 0)),
                  pl.BlockSpec((8, 1024), lambda: (0, 0)),
                  pl.BlockSpec((2048, 512), lambda: (0, 0)),
                  pl.BlockSpec((8, 512), lambda: (0, 0)),
                  pl.BlockSpec((512, 256), lambda: (0, 0)),
                  pl.BlockSpec((1, 256), lambda: (0, 0)),
                  pl.BlockSpec((8, 256), lambda: (0, 0)),
                  pl.BlockSpec((256, 40), lambda: (0, 0)),
                  pl.BlockSpec((1, 40), lambda: (0, 0))],
        out_specs=pl.BlockSpec((B, 40), lambda: (0, 0)),
        out_shape=jax.ShapeDtypeStruct((B, 40), F32),
    )(y5, st5, gbs[4], L1.T, gbs[5], L2.T, b2[None, :], gbs[6],
      L3.T, b3[None, :])

    return out


# speedup vs baseline: 1.3762x; 1.3762x over previous
import functools

import jax
import jax.numpy as jnp
from jax.experimental import pallas as pl
from jax.experimental.pallas import tpu as pltpu
from jax.experimental.pallas import tpu_sc as plsc

KNN = 20
EPSV = 1e-5
_HI = jax.lax.Precision.HIGHEST
F32 = jnp.float32


def _lrelu(v):
    return jnp.where(v >= 0, v, 0.2 * v)


def _mm(a, b):
    return jax.lax.dot_general(a, b, (((1,), (0,)), ((), ())),
                               precision=_HI, preferred_element_type=F32)


def _mmt(a, b):
    return jax.lax.dot_general(a, b, (((1,), (1,)), ((), ())),
                               precision=_HI, preferred_element_type=F32)


BF16 = jnp.bfloat16


def _mmb(a, b):
    return jax.lax.dot_general(a.astype(BF16), b.astype(BF16),
                               (((1,), (0,)), ((), ())),
                               preferred_element_type=F32)


def _mmtb(a, b):
    return jax.lax.dot_general(a.astype(BF16), b.astype(BF16),
                               (((1,), (1,)), ((), ())),
                               preferred_element_type=F32)


def _dist(xc, scol, srow, m, n):
    g = _mmtb(xc[:m], xc[:n])
    return (scol[:m] - 2.0 * g) + srow[:, :n]


def _mmbb(a, b):
    return jax.lax.dot_general(a, b, (((1,), (0,)), ((), ())),
                               preferred_element_type=F32)


def _knn_maxpool(d, feats):
    M, N = d.shape
    C = feats.shape[1]
    h1 = feats.astype(BF16)
    r1 = feats - h1.astype(F32)
    h2 = r1.astype(BF16)
    h3 = (r1 - h2.astype(F32)).astype(BF16)
    iota = jax.lax.broadcasted_iota(jnp.int32, (M, N), 1)
    pooled = jnp.full((M, C), -jnp.inf, F32)
    for t in range(KNN):
        if t < KNN - 1:
            am = jnp.argmin(d, axis=1)[:, None]
        else:
            rowmin = jnp.min(d, axis=1, keepdims=True)
            am = jnp.min(jnp.where(d == rowmin, iota, N), axis=1,
                         keepdims=True)
        onehot = iota == am
        d = jnp.where(onehot, F32(1e30), d)
        ohb = onehot.astype(BF16)
        sel = (_mmbb(ohb, h1) + _mmbb(ohb, h2)) + _mmbb(ohb, h3)
        pooled = jnp.maximum(pooled, sel)
    return pooled


def _norm(y, st, gb, cnt):
    m = st[0:1, :] / cnt
    v = st[1:2, :] / cnt - m * m
    sc = gb[0:1, :] * jax.lax.rsqrt(v + EPSV)
    return _lrelu((y - m) * sc + gb[1:2, :])


def _stat_rows(y):
    s0 = jnp.sum(y, axis=0, keepdims=True)
    s1 = jnp.sum(y * y, axis=0, keepdims=True)
    rows = jax.lax.broadcasted_iota(jnp.int32, (8, y.shape[1]), 0)
    return jnp.where(rows == 0, jnp.broadcast_to(s0, (8, y.shape[1])),
                     jnp.where(rows == 1, jnp.broadcast_to(s1, (8, y.shape[1])),
                               F32(0.0)))


def _accum_stats(b, nb, upd, st_ref, acc_ref):
    @pl.when(b == 0)
    def _():
        acc_ref[...] = jnp.zeros_like(acc_ref)
    acc_ref[...] += upd

    @pl.when(b == nb - 1)
    def _():
        st_ref[...] = acc_ref[...]


def _c1_body(nb, xd_ref, w_ref, y_ref, st_ref, acc_ref):
    b = pl.program_id(0)
    y = _mmb(xd_ref[0], w_ref[...])
    y_ref[0] = y
    _accum_stats(b, nb, _stat_rows(y), st_ref, acc_ref)


def _mid_body(nb, npts_in, npool, nsub, y_ref, st_ref, xc_ref, scol_ref,
              srow_ref, w_ref, gb_ref, yo_ref, xs_ref, st2_ref, acc_ref):
    b = pl.program_id(0)
    cnt = F32(nb * npts_in)
    f = _norm(y_ref[0], st_ref[...], gb_ref[...], cnt)
    xc = xc_ref[0]
    scol = scol_ref[0]
    srow = srow_ref[0]
    d = _dist(xc, scol, srow, npool, npts_in)
    fp = _knn_maxpool(d, f)
    ds = _dist(xc, scol, srow, nsub, npool)
    xs_ref[0] = _knn_maxpool(ds, fp)
    yo = _mmb(fp, w_ref[...])
    yo_ref[0] = yo
    _accum_stats(b, nb, _stat_rows(yo), st2_ref, acc_ref)


def _c2a_body(nb, npts, npool, y_ref, st_ref, xc_ref, scol_ref, srow_ref,
              gb_ref, f_ref, idx_ref):
    b = pl.program_id(0)
    cnt = F32(nb * npts)
    f = _norm(y_ref[0], st_ref[...], gb_ref[...], cnt)
    f_ref[0] = jnp.concatenate(
        [f, jnp.zeros((f.shape[0], 128 - f.shape[1]), F32)], axis=1)
    d = _dist(xc_ref[0], scol_ref[0], srow_ref[0], npool, npts)
    base = b * npts
    iota = jax.lax.broadcasted_iota(jnp.int32, (npool, npts), 1)
    for t in range(KNN):
        if t < KNN - 1:
            am = jnp.argmin(d, axis=1)[:, None]
        else:
            rowmin = jnp.min(d, axis=1, keepdims=True)
            am = jnp.min(jnp.where(d == rowmin, iota, npts), axis=1,
                         keepdims=True)
        idx_ref[0, :, t:t + 1] = am + base
        d = jnp.where(iota == am, F32(1e30), d)


def _sc_gather_body(per_tile, chunk, table_ref, idx_ref, out_ref,
                    idx_v, rows_v, sem):
    c = jax.lax.axis_index("c")
    s = jax.lax.axis_index("s")
    base = (s * 2 + c) * per_tile

    @pl.loop(0, per_tile, step=chunk)
    def _(off):
        pltpu.sync_copy(idx_ref.at[pl.ds(base + off, chunk)], idx_v)
        pltpu.async_copy(table_ref.at[idx_v], rows_v, sem).wait()
        pltpu.sync_copy(rows_v, out_ref.at[pl.ds(base + off, chunk)])


def _c2c_body(nb, npool, nsub, cin, cout, g_ref, xc_ref, scol_ref, srow_ref,
              w_ref, yo_ref, xs_ref, st2_ref, acc_ref):
    b = pl.program_id(0)
    g = g_ref[0]
    fp = jnp.max(g.reshape(npool, KNN, 128), axis=1)[:, :cin]
    ds = _dist(xc_ref[0], scol_ref[0], srow_ref[0], nsub, npool)
    xs_ref[0] = _knn_maxpool(ds, fp)
    yo = _mmb(fp, w_ref[...])
    yo_ref[0] = yo
    _accum_stats(b, nb, _stat_rows(yo), st2_ref, acc_ref)


def _c5_body(nb, npts_in, nsub, y_ref, st_ref, xc_ref, scol_ref, srow_ref,
             x1_ref, x2_ref, x3_ref, w_ref, gb_ref, yo_ref, st2_ref, acc_ref):
    b = pl.program_id(0)
    cnt = F32(nb * npts_in)
    f4 = _norm(y_ref[0], st_ref[...], gb_ref[...], cnt)
    d4 = _dist(xc_ref[0], scol_ref[0], srow_ref[0], nsub, npts_in)
    fp4 = _knn_maxpool(d4, f4)
    h = jnp.concatenate([x1_ref[0], x2_ref[0], x3_ref[0], fp4], axis=1)
    yo = _mmb(h, w_ref[...])
    yo_ref[0] = yo
    _accum_stats(b, nb, _stat_rows(yo), st2_ref, acc_ref)


def _head_body(nb, nsub, y5_ref, st_ref, gb5_ref, l1_ref, gb6_ref, l2_ref,
               b2_ref, gb7_ref, l3_ref, b3_ref, o_ref):
    cnt = F32(nb * nsub)
    y5 = y5_ref[...]
    st = st_ref[...]
    m = (st[0:1, :] / cnt)[None]
    v = (st[1:2, :] / cnt)[None] - m * m
    sc = gb5_ref[0:1, :][None] * jax.lax.rsqrt(v + EPSV)
    f5 = _lrelu((y5 - m) * sc + gb5_ref[1:2, :][None])
    h1 = jnp.max(f5, axis=1)
    h2 = jnp.mean(f5, axis=1)
    h = jnp.concatenate([h1, h2], axis=1)

    t = _mmb(h, l1_ref[...])
    mb = jnp.mean(t, axis=0, keepdims=True)
    vb = jnp.mean(t * t, axis=0, keepdims=True) - mb * mb
    t = _lrelu(gb6_ref[0:1, :] * (t - mb) * jax.lax.rsqrt(vb + EPSV)
               + gb6_ref[1:2, :])

    t = _mmb(t, l2_ref[...]) + b2_ref[...]
    mb = jnp.mean(t, axis=0, keepdims=True)
    vb = jnp.mean(t * t, axis=0, keepdims=True) - mb * mb
    t = _lrelu(gb7_ref[0:1, :] * (t - mb) * jax.lax.rsqrt(vb + EPSV)
               + gb7_ref[1:2, :])

    o_ref[...] = _mmb(t, l3_ref[...]) + b3_ref[...]


def _pack_gb(g, be):
    z = jnp.zeros((8, g.shape[0]), F32)
    return z.at[0].set(g).at[1].set(be)


def kernel(x, W1, W2, W3, W4, W5, L1, L2, b2, L3, b3,
           g1, be1, g2, be2, g3, be3, g4, be4, g5, be5, g6, be6, g7, be7):
    B, _, N = x.shape
    xt = jnp.transpose(x, (0, 2, 1))
    scol = jnp.sum(xt * xt, axis=2, keepdims=True)
    srow = jnp.transpose(scol, (0, 2, 1))
    xc = jnp.concatenate([xt, jnp.zeros((B, N, 5), F32)], axis=2)

    w1p = jnp.zeros((8, 64), F32).at[:3].set(W1.T)
    gbs = [_pack_gb(g1, be1), _pack_gb(g2, be2), _pack_gb(g3, be3),
           _pack_gb(g4, be4), _pack_gb(g5, be5), _pack_gb(g6, be6),
           _pack_gb(g7, be7)]

    cparams = pltpu.CompilerParams(dimension_semantics=("arbitrary",))

    per_b = lambda *dims: pl.BlockSpec((1,) + dims, lambda b: (b, 0, 0))
    const2 = lambda r, c: pl.BlockSpec((r, c), lambda b: (0, 0))

    y1, st1 = pl.pallas_call(
        functools.partial(_c1_body, B),
        grid=(B,),
        in_specs=[per_b(N, 8), const2(8, 64)],
        out_specs=[per_b(N, 64), const2(8, 64)],
        out_shape=[jax.ShapeDtypeStruct((B, N, 64), F32),
                   jax.ShapeDtypeStruct((8, 64), F32)],
        scratch_shapes=[pltpu.VMEM((8, 64), F32)],
        compiler_params=cparams,
    )(xc, w1p)

    def mid_call(yin, stin, cin, cout, npts_in, npool, nsub, wt, gb):
        return pl.pallas_call(
            functools.partial(_mid_body, B, npts_in, npool, nsub),
            grid=(B,),
            in_specs=[per_b(npts_in, cin), const2(8, cin),
                      per_b(npts_in, 8), per_b(npts_in, 1),
                      pl.BlockSpec((1, 1, npts_in), lambda b: (b, 0, 0)),
                      const2(cin, cout), const2(8, cin)],

            out_specs=[per_b(npool, cout), per_b(nsub, cin), const2(8, cout)],
            out_shape=[jax.ShapeDtypeStruct((B, npool, cout), F32),
                       jax.ShapeDtypeStruct((B, nsub, cin), F32),
                       jax.ShapeDtypeStruct((8, cout), F32)],
            scratch_shapes=[pltpu.VMEM((8, cout), F32)],
            compiler_params=cparams,
        )(yin, stin, xc[:, :npts_in], scol[:, :npts_in],
          srow[:, :, :npts_in], wt, gb)

    npool1 = N // 4
    hb = B // 2

    def stage1_half(y1h, xch, scolh, srowh):
        f1h, idxh = pl.pallas_call(
            functools.partial(_c2a_body, B, N, npool1),
            grid=(hb,),
            in_specs=[per_b(N, 64), const2(8, 64),
                      per_b(N, 8), per_b(N, 1),
                      pl.BlockSpec((1, 1, N), lambda b: (b, 0, 0)),
                      const2(8, 64)],
            out_specs=[per_b(N, 128), per_b(npool1, KNN)],
            out_shape=[jax.ShapeDtypeStruct((hb, N, 128), F32),
                       jax.ShapeDtypeStruct((hb, npool1, KNN), jnp.int32)],
            compiler_params=cparams,
        )(y1h, st1, xch, scolh, srowh, gbs[0])

        tot = hb * npool1 * KNN
        per_tile = tot // 32
        chunk = 256
        gath = pl.kernel(
            functools.partial(_sc_gather_body, per_tile, chunk),
            out_type=jax.ShapeDtypeStruct((tot, 128), F32),
            mesh=plsc.VectorSubcoreMesh(core_axis_name="c",
                                        subcore_axis_name="s"),
            scratch_types=[pltpu.VMEM((chunk,), jnp.int32),
                           pltpu.VMEM((chunk, 128), F32),
                           pltpu.SemaphoreType.DMA],
        )(f1h.reshape(hb * N, 128), idxh.reshape(tot))
        return gath

    def stage1_back(gath, xch, scolh, srowh):
        return pl.pallas_call(
            functools.partial(_c2c_body, hb, npool1, N // 64, 64, 64),
            grid=(hb,),
            in_specs=[per_b(npool1 * KNN, 128),
                      per_b(npool1, 8), per_b(npool1, 1),
                      pl.BlockSpec((1, 1, npool1), lambda b: (b, 0, 0)),
                      const2(64, 64)],
            out_specs=[per_b(npool1, 64), per_b(N // 64, 64), const2(8, 64)],
            out_shape=[jax.ShapeDtypeStruct((hb, npool1, 64), F32),
                       jax.ShapeDtypeStruct((hb, N // 64, 64), F32),
                       jax.ShapeDtypeStruct((8, 64), F32)],
            scratch_shapes=[pltpu.VMEM((8, 64), F32)],
            compiler_params=cparams,
        )(gath.reshape(hb, npool1 * KNN, 128), xch,
          scolh, srowh, W2.T)

    g_h1 = stage1_half(y1[:hb], xc[:hb], scol[:hb], srow[:hb])
    g_h2 = stage1_half(y1[hb:], xc[hb:], scol[hb:], srow[hb:])
    y2a, x1a, st2a = stage1_back(g_h1, xc[:hb, :npool1],
                                 scol[:hb, :npool1], srow[:hb, :, :npool1])
    y2b, x1b, st2b = stage1_back(g_h2, xc[hb:, :npool1],
                                 scol[hb:, :npool1], srow[hb:, :, :npool1])
    y2 = jnp.concatenate([y2a, y2b], axis=0)
    x1 = jnp.concatenate([x1a, x1b], axis=0)
    st2 = st2a + st2b
    y3, x2, st3 = mid_call(y2, st2, 64, 128, N // 4, N // 16, N // 64,
                           W3.T, gbs[1])
    y4, x3, st4 = mid_call(y3, st3, 128, 256, N // 16, N // 32, N // 64,
                           W4.T, gbs[2])

    p5 = N // 32
    ns = N // 64
    y5, st5 = pl.pallas_call(
        functools.partial(_c5_body, B, p5, ns),
        grid=(B,),
        in_specs=[per_b(p5, 256), const2(8, 256),
                  per_b(p5, 8), per_b(p5, 1),
                  pl.BlockSpec((1, 1, p5), lambda b: (b, 0, 0)),
                  per_b(ns, 64), per_b(ns, 64), per_b(ns, 128),
                  const2(512, 1024), const2(8, 256)],
        out_specs=[per_b(ns, 1024), const2(8, 1024)],
        out_shape=[jax.ShapeDtypeStruct((B, ns, 1024), F32),
                   jax.ShapeDtypeStruct((8, 1024), F32)],
        scratch_shapes=[pltpu.VMEM((8, 1024), F32)],
        compiler_params=cparams,
    )(y4, st4, xc[:, :p5], scol[:, :p5], srow[:, :, :p5],
      x1, x2, x3, W5.T, gbs[3])

    out = pl.pallas_call(
        functools.partial(_head_body, B, ns),
        in_specs=[pl.BlockSpec((B, ns, 1024), lambda: (0, 0, 0)),
                  pl.BlockSpec((8, 1024), lambda: (0, 0)),
                  pl.BlockSpec((8, 1024), lambda: (0, 0)),
                  pl.BlockSpec((2048, 512), lambda: (0, 0)),
                  pl.BlockSpec((8, 512), lambda: (0, 0)),
                  pl.BlockSpec((512, 256), lambda: (0, 0)),
                  pl.BlockSpec((1, 256), lambda: (0, 0)),
                  pl.BlockSpec((8, 256), lambda: (0, 0)),
                  pl.BlockSpec((256, 40), lambda: (0, 0)),
                  pl.BlockSpec((1, 40), lambda: (0, 0))],
        out_specs=pl.BlockSpec((B, 40), lambda: (0, 0)),
        out_shape=jax.ShapeDtypeStruct((B, 40), F32),
    )(y5, st5, gbs[4], L1.T, gbs[5], L2.T, b2[None, :], gbs[6],
      L3.T, b3[None, :])

    return out
